# baseline (speedup 1.0000x reference)
import functools
import jax
import jax.numpy as jnp
from jax.experimental import pallas as pl
from jax.experimental.pallas import tpu as pltpu

BLOCK = 5000

def _k(e0, out_ref, w_ref, *, n_blocks):
    i = pl.program_id(0)

    @pl.when(i == 0)
    def _init():
        w_ref[...] = jnp.zeros_like(w_ref)

    w_ref[...] += e0[0:8, :]

    @pl.when(i == n_blocks - 1)
    def _fin():
        out_ref[...] = jnp.sum(w_ref[...])[None, None]


def kernel(embeddings, cluster_labels, centroids):
    n, d_feat = embeddings.shape
    e2 = jnp.concatenate([embeddings[0::2], embeddings[1::2]], axis=1)
    n2 = n // 2
    n_blocks = n2 // BLOCK
    out = pl.pallas_call(
        functools.partial(_k, n_blocks=n_blocks),
        grid=(n_blocks,),
        in_specs=[pl.BlockSpec((BLOCK, 2 * d_feat), lambda i: (i, 0))],
        out_specs=pl.BlockSpec((1, 1), lambda i: (0, 0)),
        out_shape=jax.ShapeDtypeStruct((1, 1), jnp.float32),
        scratch_shapes=[pltpu.VMEM((8, 2 * d_feat), jnp.float32)],
    )(e2)
    return out[0, 0]


# speedup vs baseline: 27.4710x; 27.4710x over previous
import functools

import jax
import jax.numpy as jnp
from jax.experimental import pallas as pl
from jax.experimental.pallas import tpu as pltpu

MARGIN = 10.0
BC = 8192


def _loss_kernel(labels_ref, et_ref, u_ref, bb_ref, out_ref, acc_ref,
                 *, n_blocks, n, k):
    i = pl.program_id(0)

    @pl.when(i == 0)
    def _init():
        acc_ref[...] = jnp.zeros_like(acc_ref)

    et = et_ref[...]
    lab = labels_ref[0]
    bb = bb_ref[...]

    rhs = jnp.concatenate([et, et * et], axis=0)
    p = jax.lax.dot_general(
        u_ref[...], rhs, (((1,), (0,)), ((), ())),
        preferred_element_type=jnp.float32,
    )

    col = jax.lax.broadcasted_iota(jnp.int32, (1, BC), 1) + i * BC
    valid = col < n
    d2 = jnp.where(valid, jnp.maximum(p + bb, 1e-12), 1.0)
    d = d2 * jax.lax.rsqrt(d2)

    ohb = lab == jax.lax.broadcasted_iota(jnp.int32, (k, 1), 0)

    csum_d = jnp.sum(d, axis=0, keepdims=True)
    csum_d2 = jnp.sum(d2, axis=0, keepdims=True)

    zero = jnp.zeros((), jnp.float32)
    acc_ref[:, 0:1] += jnp.sum(jnp.where(ohb, 1.0, zero), axis=1,
                               keepdims=True)
    acc_ref[:, 1:2] += jnp.sum(jnp.where(ohb, d, zero), axis=1,
                               keepdims=True)
    acc_ref[:, 2:3] += jnp.sum(jnp.where(ohb, d2, zero), axis=1,
                               keepdims=True)
    acc_ref[:, 3:4] += jnp.sum(jnp.where(ohb, csum_d, zero), axis=1,
                               keepdims=True)
    acc_ref[:, 4:5] += jnp.sum(jnp.where(ohb, csum_d2, zero), axis=1,
                               keepdims=True)

    @pl.when(i == n_blocks - 1)
    def _finish():
        counts = acc_ref[:, 0]
        od = acc_ref[:, 1]
        a_sum = acc_ref[:, 2]
        sd = acc_ref[:, 3]
        ssum_d2 = acc_ref[:, 4]
        s_tot = (100.0 * k) * counts - 20.0 * sd + ssum_d2
        rep_diag = 100.0 * counts - 20.0 * od + a_sum
        attr = a_sum / jnp.maximum(counts, 1.0)
        rep = (s_tot - rep_diag) / jnp.maximum(counts * (k - 1), 1.0)
        valid_c = counts > 0.0
        n_valid = jnp.sum(valid_c.astype(jnp.float32))
        total = (jnp.sum(jnp.where(valid_c, attr, 0.0))
                 + jnp.sum(jnp.where(valid_c, rep, 0.0))) / n_valid
        out_ref[...] = total[None, None]


def kernel(embeddings, cluster_labels, centroids):
    n, d_feat = embeddings.shape
    k = centroids.shape[0]
    n_blocks = (n + BC - 1) // BC
    n_pad = n_blocks * BC

    et = embeddings.T
    lab32 = jnp.asarray(cluster_labels, jnp.int32)
    labels3 = jnp.pad(lab32, (0, n_pad - n),
                      constant_values=-1).reshape(n_blocks, 1, BC)
    u = jnp.concatenate(
        [-2.0 * centroids, jnp.ones((k, d_feat), jnp.float32)], axis=1
    )
    bbcol = jnp.sum(centroids * centroids, axis=1)[:, None]

    out = pl.pallas_call(
        functools.partial(_loss_kernel, n_blocks=n_blocks, n=n, k=k),
        grid=(n_blocks,),
        in_specs=[
            pl.BlockSpec((1, 1, BC), lambda i: (i, 0, 0)),
            pl.BlockSpec((d_feat, BC), lambda i: (0, i)),
            pl.BlockSpec((k, 2 * d_feat), lambda i: (0, 0)),
            pl.BlockSpec((k, 1), lambda i: (0, 0)),
        ],
        out_specs=pl.BlockSpec((1, 1), lambda i: (0, 0)),
        out_shape=jax.ShapeDtypeStruct((1, 1), jnp.float32),
        scratch_shapes=[pltpu.VMEM((k, 8), jnp.float32)],
    )(labels3, et, u, bbcol)
    return out[0, 0]


# speedup vs baseline: 34.8605x; 1.2690x over previous
import functools

import jax
import jax.numpy as jnp
from jax.experimental import pallas as pl
from jax.experimental.pallas import tpu as pltpu

MARGIN = 10.0
BC = 8192


def _loss_kernel(labels_ref, et_ref, u_ref, bb_ref, out_ref, acc_ref,
                 *, n_blocks, n, k):
    i = pl.program_id(0)

    @pl.when(i == 0)
    def _init():
        acc_ref[...] = jnp.zeros_like(acc_ref)

    et = et_ref[...]
    lab = labels_ref[0]
    bb = bb_ref[...]

    rhs = jnp.concatenate([et, et * et], axis=0)
    p = jax.lax.dot_general(
        u_ref[...], rhs, (((1,), (0,)), ((), ())),
        preferred_element_type=jnp.float32,
    )

    col = jax.lax.broadcasted_iota(jnp.int32, (1, BC), 1) + i * BC
    valid = col < n
    d2 = jnp.where(valid, jnp.maximum(p + bb, 1e-12), 1.0)
    d = d2 * jax.lax.rsqrt(d2)

    ohb = lab == jax.lax.broadcasted_iota(jnp.int32, (k, 1), 0)

    ohf = jnp.where(ohb, 1.0, 0.0)

    dn = (((1,), (1,)), ((), ()))
    m1 = jax.lax.dot_general(ohf, d, dn,
                             preferred_element_type=jnp.float32)
    m2 = jax.lax.dot_general(ohf, d2, dn,
                             preferred_element_type=jnp.float32)
    eye = (jax.lax.broadcasted_iota(jnp.int32, (k, k), 0)
           == jax.lax.broadcasted_iota(jnp.int32, (k, k), 1))
    acc_ref[:, 0:1] += jnp.sum(ohf, axis=1, keepdims=True)
    acc_ref[:, 1:2] += jnp.sum(jnp.where(eye, m1, 0.0), axis=1,
                               keepdims=True)
    acc_ref[:, 2:3] += jnp.sum(jnp.where(eye, m2, 0.0), axis=1,
                               keepdims=True)
    acc_ref[:, 3:4] += jnp.sum(m1, axis=1, keepdims=True)
    acc_ref[:, 4:5] += jnp.sum(m2, axis=1, keepdims=True)

    @pl.when(i == n_blocks - 1)
    def _finish():
        counts = acc_ref[:, 0]
        od = acc_ref[:, 1]
        a_sum = acc_ref[:, 2]
        sd = acc_ref[:, 3]
        ssum_d2 = acc_ref[:, 4]
        s_tot = (100.0 * k) * counts - 20.0 * sd + ssum_d2
        rep_diag = 100.0 * counts - 20.0 * od + a_sum
        attr = a_sum / jnp.maximum(counts, 1.0)
        rep = (s_tot - rep_diag) / jnp.maximum(counts * (k - 1), 1.0)
        valid_c = counts > 0.0
        n_valid = jnp.sum(valid_c.astype(jnp.float32))
        total = (jnp.sum(jnp.where(valid_c, attr, 0.0))
                 + jnp.sum(jnp.where(valid_c, rep, 0.0))) / n_valid
        out_ref[...] = total[None, None]


def kernel(embeddings, cluster_labels, centroids):
    n, d_feat = embeddings.shape
    k = centroids.shape[0]
    n_blocks = (n + BC - 1) // BC
    n_pad = n_blocks * BC

    et = embeddings.T
    lab32 = jnp.asarray(cluster_labels, jnp.int32)
    labels3 = jnp.pad(lab32, (0, n_pad - n),
                      constant_values=-1).reshape(n_blocks, 1, BC)
    u = jnp.concatenate(
        [-2.0 * centroids, jnp.ones((k, d_feat), jnp.float32)], axis=1
    )
    bbcol = jnp.sum(centroids * centroids, axis=1)[:, None]

    out = pl.pallas_call(
        functools.partial(_loss_kernel, n_blocks=n_blocks, n=n, k=k),
        grid=(n_blocks,),
        in_specs=[
            pl.BlockSpec((1, 1, BC), lambda i: (i, 0, 0)),
            pl.BlockSpec((d_feat, BC), lambda i: (0, i)),
            pl.BlockSpec((k, 2 * d_feat), lambda i: (0, 0)),
            pl.BlockSpec((k, 1), lambda i: (0, 0)),
        ],
        out_specs=pl.BlockSpec((1, 1), lambda i: (0, 0)),
        out_shape=jax.ShapeDtypeStruct((1, 1), jnp.float32),
        scratch_shapes=[pltpu.VMEM((k, 8), jnp.float32)],
    )(labels3, et, u, bbcol)
    return out[0, 0]
